# baseline (speedup 1.0000x reference)
import functools

import jax
import jax.numpy as jnp
from jax import lax
from jax.experimental import pallas as pl
from jax.experimental.pallas import tpu as pltpu
from jax.experimental.pallas import tpu_sc as plsc

_NC = 2
_NS = 16
_NW = _NC * _NS
_C = 32


def _embed_sc(ids3, table):
    nw, nchunk, c = ids3.shape
    total = nw * nchunk * c
    d = table.shape[1]
    mesh = plsc.VectorSubcoreMesh(
        core_axis_name="c", subcore_axis_name="s",
        num_cores=_NC, num_subcores=_NS)

    @functools.partial(
        pl.kernel,
        out_type=jax.ShapeDtypeStruct((total, d), jnp.float32),
        mesh=mesh,
        scratch_types=[
            pltpu.VMEM((nchunk, c), jnp.int32),
            pltpu.VMEM((c, d), jnp.float32),
            pltpu.VMEM((c, d), jnp.float32),
            pltpu.VMEM((c, d), jnp.float32),
            pltpu.SemaphoreType.DMA,
            pltpu.SemaphoreType.DMA,
            pltpu.SemaphoreType.DMA,
            pltpu.SemaphoreType.DMA,
            pltpu.SemaphoreType.DMA,
            pltpu.SemaphoreType.DMA,
        ],
    )
    def k(ids_hbm, table_hbm, out_hbm, idx_v,
          buf0, buf1, buf2, gs0, gs1, gs2, ws0, ws1, ws2):
        wid = lax.axis_index("s") * _NC + lax.axis_index("c")
        base = wid * (nchunk * c)
        pltpu.sync_copy(ids_hbm.at[wid], idx_v)
        nbuf = 3
        bufs = (buf0, buf1, buf2)
        gsems = (gs0, gs1, gs2)
        wsems = (ws0, ws1, ws2)
        gcps = [None] * nbuf
        wcps = [None] * nbuf
        for j in range(nbuf):
            gcps[j] = pltpu.async_copy(
                table_hbm.at[idx_v.at[j]], bufs[j], gsems[j])
        for j in range(nchunk):
            cur = j % nbuf
            gcps[cur].wait()
            wcps[cur] = pltpu.async_copy(
                bufs[cur], out_hbm.at[pl.ds(base + j * c, c)], wsems[cur])
            nj = j + nbuf
            if nj < nchunk:
                wcps[cur].wait()
                gcps[cur] = pltpu.async_copy(
                    table_hbm.at[idx_v.at[nj]], bufs[cur], gsems[cur])
        for j in range(nchunk - nbuf, nchunk):
            wcps[j % nbuf].wait()

    return k(ids3, table)


def kernel(input_ids, embed_table):
    b, s = input_ids.shape
    d = embed_table.shape[1]
    total = b * s
    nchunk = total // (_NW * _C)
    ids3 = input_ids.reshape(_NW, nchunk, _C).astype(jnp.int32)
    out = _embed_sc(ids3, embed_table.astype(jnp.float32))
    return out.reshape(b, s, d)


# speedup vs baseline: 1.0018x; 1.0018x over previous
import functools

import jax
import jax.numpy as jnp
from jax import lax
from jax.experimental import pallas as pl
from jax.experimental.pallas import tpu as pltpu
from jax.experimental.pallas import tpu_sc as plsc

_NC = 2
_NS = 16
_NW = _NC * _NS
_C = 32


def kernel(input_ids, embed_table):
    b, s = input_ids.shape
    d = embed_table.shape[1]
    per_w = (b * s) // _NW
    nchunk = per_w // _C
    w_per_b = s // per_w
    mesh = plsc.VectorSubcoreMesh(
        core_axis_name="c", subcore_axis_name="s",
        num_cores=_NC, num_subcores=_NS)

    @functools.partial(
        pl.kernel,
        out_type=jax.ShapeDtypeStruct((b, s, d), jnp.float32),
        mesh=mesh,
        scratch_types=[
            pltpu.VMEM((per_w,), jnp.int32),
            pltpu.VMEM((_C, d), jnp.float32),
            pltpu.VMEM((_C, d), jnp.float32),
            pltpu.VMEM((_C, d), jnp.float32),
            pltpu.SemaphoreType.DMA,
            pltpu.SemaphoreType.DMA,
            pltpu.SemaphoreType.DMA,
            pltpu.SemaphoreType.DMA,
            pltpu.SemaphoreType.DMA,
            pltpu.SemaphoreType.DMA,
        ],
    )
    def k(ids_hbm, table_hbm, out_hbm, idx_v,
          buf0, buf1, buf2, gs0, gs1, gs2, ws0, ws1, ws2):
        wid = lax.axis_index("s") * _NC + lax.axis_index("c")
        row = wid // w_per_b
        off = (wid % w_per_b) * per_w
        pltpu.sync_copy(ids_hbm.at[row, pl.ds(off, per_w)], idx_v)
        nbuf = 3
        bufs = (buf0, buf1, buf2)
        gsems = (gs0, gs1, gs2)
        wsems = (ws0, ws1, ws2)
        gcps = [None] * nbuf
        wcps = [None] * nbuf
        for j in range(nbuf):
            gcps[j] = pltpu.async_copy(
                table_hbm.at[idx_v.at[pl.ds(j * _C, _C)]], bufs[j], gsems[j])
        for j in range(nchunk):
            cur = j % nbuf
            gcps[cur].wait()
            wcps[cur] = pltpu.async_copy(
                bufs[cur], out_hbm.at[row, pl.ds(off + j * _C, _C)],
                wsems[cur])
            nj = j + nbuf
            if nj < nchunk:
                wcps[cur].wait()
                gcps[cur] = pltpu.async_copy(
                    table_hbm.at[idx_v.at[pl.ds(nj * _C, _C)]],
                    bufs[cur], gsems[cur])
        for j in range(nchunk - nbuf, nchunk):
            wcps[j % nbuf].wait()

    return k(input_ids.astype(jnp.int32), embed_table)


# speedup vs baseline: 1.0121x; 1.0103x over previous
import functools

import jax
import jax.numpy as jnp
from jax import lax
from jax.experimental import pallas as pl
from jax.experimental.pallas import tpu as pltpu
from jax.experimental.pallas import tpu_sc as plsc

_NC = 2
_NS = 16
_NW = _NC * _NS
_C = 16


def kernel(input_ids, embed_table):
    b, s = input_ids.shape
    d = embed_table.shape[1]
    per_w = (b * s) // _NW
    nchunk = per_w // _C
    w_per_b = s // per_w
    mesh = plsc.VectorSubcoreMesh(
        core_axis_name="c", subcore_axis_name="s",
        num_cores=_NC, num_subcores=_NS)

    @functools.partial(
        pl.kernel,
        out_type=jax.ShapeDtypeStruct((b, s, d), jnp.float32),
        mesh=mesh,
        scratch_types=[
            pltpu.VMEM((per_w,), jnp.int32),
            pltpu.VMEM((_C, d), jnp.float32),
            pltpu.VMEM((_C, d), jnp.float32),
            pltpu.VMEM((_C, d), jnp.float32),
            pltpu.VMEM((_C, d), jnp.float32),
            pltpu.VMEM((_C, d), jnp.float32),
            pltpu.VMEM((_C, d), jnp.float32),
            pltpu.VMEM((_C, d), jnp.float32),
            pltpu.SemaphoreType.DMA,
            pltpu.SemaphoreType.DMA,
            pltpu.SemaphoreType.DMA,
            pltpu.SemaphoreType.DMA,
            pltpu.SemaphoreType.DMA,
            pltpu.SemaphoreType.DMA,
            pltpu.SemaphoreType.DMA,
            pltpu.SemaphoreType.DMA,
            pltpu.SemaphoreType.DMA,
            pltpu.SemaphoreType.DMA,
            pltpu.SemaphoreType.DMA,
            pltpu.SemaphoreType.DMA,
            pltpu.SemaphoreType.DMA,
            pltpu.SemaphoreType.DMA,
        ],
    )
    def k(ids_hbm, table_hbm, out_hbm, idx_v,
          buf0, buf1, buf2, buf3, buf4, buf5, buf6,
          gs0, gs1, gs2, gs3, gs4, gs5, gs6,
          ws0, ws1, ws2, ws3, ws4, ws5, ws6):
        wid = lax.axis_index("s") * _NC + lax.axis_index("c")
        row = wid // w_per_b
        off = (wid % w_per_b) * per_w
        pltpu.sync_copy(ids_hbm.at[row, pl.ds(off, per_w)], idx_v)
        nbuf = 7
        bufs = (buf0, buf1, buf2, buf3, buf4, buf5, buf6)
        gsems = (gs0, gs1, gs2, gs3, gs4, gs5, gs6)
        wsems = (ws0, ws1, ws2, ws3, ws4, ws5, ws6)
        gcps = [None] * nbuf
        wcps = [None] * nbuf
        for j in range(nbuf):
            gcps[j] = pltpu.async_copy(
                table_hbm.at[idx_v.at[pl.ds(j * _C, _C)]], bufs[j], gsems[j])
        for j in range(nchunk):
            cur = j % nbuf
            gcps[cur].wait()
            wcps[cur] = pltpu.async_copy(
                bufs[cur], out_hbm.at[row, pl.ds(off + j * _C, _C)],
                wsems[cur])
            nj = j + nbuf
            if nj < nchunk:
                wcps[cur].wait()
                gcps[cur] = pltpu.async_copy(
                    table_hbm.at[idx_v.at[pl.ds(nj * _C, _C)]],
                    bufs[cur], gsems[cur])
        for j in range(nchunk - nbuf, nchunk):
            wcps[j % nbuf].wait()

    return k(input_ids.astype(jnp.int32), embed_table)
